# baseline (speedup 1.0000x reference)
import jax
import jax.numpy as jnp
from jax.experimental import pallas as pl
from jax.experimental.pallas import tpu as pltpu

_LONG, _LAT = 128, 128
_N = _LONG * _LAT
_B = 64
_OUT = 256
_OFFSETS = (-1, 1, _LAT, -_LAT, _LAT - 1, _LAT + 1, -_LAT - 1, -_LAT + 1)
_LO, _HI = _LAT + 1, (_LONG - 1) * _LAT - 2
_NS = 4
_RC = _OUT // _NS


def _gnn_kernel(x_hbm, w_hbm, b_ref, out_ref, xv_ref, wv_ref, h1_ref, sems):
    pltpu.make_async_copy(x_hbm, xv_ref, sems.at[_NS]).start()
    for i in range(_NS):
        pltpu.make_async_copy(
            w_hbm.at[i * _RC:(i + 1) * _RC, :],
            wv_ref.at[i * _RC:(i + 1) * _RC, :],
            sems.at[i],
        ).start()
    pltpu.make_async_copy(x_hbm, xv_ref, sems.at[_NS]).wait()

    idx = jax.lax.broadcasted_iota(jnp.int32, (1, _N), 1)
    m0 = ((idx >= _LO) & (idx <= _HI)).astype(jnp.float32)
    xv = xv_ref[...]
    h = xv
    for o in _OFFSETS:
        mo = ((idx + o >= _LO) & (idx + o <= _HI)).astype(jnp.float32)
        h = h + (m0 + mo) * pltpu.roll(xv, (-o) % _N, 1)
    h1_ref[...] = h

    for i in range(_NS):
        pltpu.make_async_copy(
            w_hbm.at[i * _RC:(i + 1) * _RC, :],
            wv_ref.at[i * _RC:(i + 1) * _RC, :],
            sems.at[i],
        ).wait()
        p = jax.lax.dot_general(
            h1_ref[...],
            wv_ref[i * _RC:(i + 1) * _RC, :],
            (((1,), (1,)), ((), ())),
            preferred_element_type=jnp.float32)
        out_ref[:, i * _RC:(i + 1) * _RC] = p + b_ref[:, i * _RC:(i + 1) * _RC]


def kernel(x, W, b):
    xf = x.reshape(_B, _N)
    b2 = b.reshape(1, _OUT)
    return pl.pallas_call(
        _gnn_kernel,
        in_specs=[
            pl.BlockSpec(memory_space=pltpu.MemorySpace.HBM),
            pl.BlockSpec(memory_space=pltpu.MemorySpace.HBM),
            pl.BlockSpec(memory_space=pltpu.MemorySpace.VMEM),
        ],
        out_specs=pl.BlockSpec(memory_space=pltpu.MemorySpace.VMEM),
        out_shape=jax.ShapeDtypeStruct((_B, _OUT), jnp.float32),
        scratch_shapes=[
            pltpu.VMEM((_B, _N), jnp.float32),
            pltpu.VMEM((_OUT, _N), jnp.float32),
            pltpu.VMEM((_B, _N), jnp.float32),
            pltpu.SemaphoreType.DMA((_NS + 1,)),
        ],
    )(xf, W, b2)


# speedup vs baseline: 1.0637x; 1.0637x over previous
import jax
import jax.numpy as jnp
from jax.experimental import pallas as pl
from jax.experimental.pallas import tpu as pltpu

_LONG, _LAT = 128, 128
_N = _LONG * _LAT
_B = 64
_OUT = 256
_OFFSETS = (-1, 1, _LAT, -_LAT, _LAT - 1, _LAT + 1, -_LAT - 1, -_LAT + 1)
_LO, _HI = _LAT + 1, (_LONG - 1) * _LAT - 2


def _gnn_kernel(xf_ref, w_ref, b_ref, out_ref, h1_ref):
    idx = jax.lax.broadcasted_iota(jnp.int32, (1, _N), 1)
    m0 = ((idx >= _LO) & (idx <= _HI)).astype(jnp.float32)
    xv = xf_ref[...]
    h = xv
    for o in _OFFSETS:
        mo = ((idx + o >= _LO) & (idx + o <= _HI)).astype(jnp.float32)
        h = h + (m0 + mo) * pltpu.roll(xv, (-o) % _N, 1)
    h1_ref[...] = h

    out_ref[...] = jax.lax.dot_general(
        h1_ref[...], w_ref[...], (((1,), (1,)), ((), ())),
        preferred_element_type=jnp.float32) + b_ref[...]


def kernel(x, W, b):
    xf = x.reshape(_B, _N)
    b2 = b.reshape(1, _OUT)
    return pl.pallas_call(
        _gnn_kernel,
        in_specs=[
            pl.BlockSpec(memory_space=pltpu.MemorySpace.VMEM),
            pl.BlockSpec(memory_space=pltpu.MemorySpace.VMEM),
            pl.BlockSpec(memory_space=pltpu.MemorySpace.VMEM),
        ],
        out_specs=pl.BlockSpec(memory_space=pltpu.MemorySpace.VMEM),
        out_shape=jax.ShapeDtypeStruct((_B, _OUT), jnp.float32),
        scratch_shapes=[pltpu.VMEM((_B, _N), jnp.float32)],
    )(xf, W, b2)


# speedup vs baseline: 1.0758x; 1.0114x over previous
import jax
import jax.numpy as jnp
from jax.experimental import pallas as pl
from jax.experimental.pallas import tpu as pltpu

_LONG, _LAT = 128, 128
_N = _LONG * _LAT
_B = 64
_OUT = 256
_OFFSETS = (-1, 1, _LAT, -_LAT, _LAT - 1, _LAT + 1, -_LAT - 1, -_LAT + 1)
_LO, _HI = _LAT + 1, (_LONG - 1) * _LAT - 2


def _gnn_kernel(xf_ref, w_ref, b_ref, out_ref):
    idx = jax.lax.broadcasted_iota(jnp.int32, (1, _N), 1)
    m0 = ((idx >= _LO) & (idx <= _HI)).astype(jnp.float32)
    xv = xf_ref[...]
    h = xv
    for o in _OFFSETS:
        mo = ((idx + o >= _LO) & (idx + o <= _HI)).astype(jnp.float32)
        h = h + (m0 + mo) * pltpu.roll(xv, (-o) % _N, 1)
    out_ref[...] = jax.lax.dot_general(
        h, w_ref[...], (((1,), (1,)), ((), ())),
        preferred_element_type=jnp.float32) + b_ref[...]


def kernel(x, W, b):
    xf = x.reshape(_B, _N)
    b2 = b.reshape(1, _OUT)
    return pl.pallas_call(
        _gnn_kernel,
        in_specs=[
            pl.BlockSpec(memory_space=pltpu.MemorySpace.VMEM),
            pl.BlockSpec(memory_space=pltpu.MemorySpace.VMEM),
            pl.BlockSpec(memory_space=pltpu.MemorySpace.VMEM),
        ],
        out_specs=pl.BlockSpec(memory_space=pltpu.MemorySpace.VMEM),
        out_shape=jax.ShapeDtypeStruct((_B, _OUT), jnp.float32),
    )(xf, W, b2)
